# baseline (speedup 1.0000x reference)
import jax
import jax.numpy as jnp
from jax.experimental import pallas as pl
from jax.experimental.pallas import tpu as pltpu

TAU = 0.07
EPS = 1e-12

B = 4
C = 512
K = 21
HW = 64 * 64
T_PX = 1024
NT = HW // T_PX


def _body(feat_ref, gt_ref, out_ref, fn_scr, k0_scr, k0n_scr, acc_ref):
    phase = pl.program_id(0)
    b = pl.program_id(1)
    t = pl.program_id(2)
    blk = b * NT + t

    @pl.when((phase == 0) & (b == 0) & (t == 0))
    def _init():
        k0_scr[...] = jnp.zeros_like(k0_scr)
        acc_ref[0] = 0.0
        acc_ref[1] = 0.0

    mask = (gt_ref[0] == 1)
    maskf = mask.astype(jnp.float32)

    @pl.when(phase == 0)
    def _phase0():
        x = feat_ref[0]
        s2 = jnp.sum(x * x, axis=0, keepdims=True)
        rnorm = 1.0 / jnp.maximum(jnp.sqrt(s2), EPS)
        fnb = (x * rnorm).astype(jnp.bfloat16)
        fn_scr[pl.ds(blk, 1)] = fnb[None]
        part = jax.lax.dot_general(
            maskf.astype(jnp.bfloat16), fnb,
            dimension_numbers=(((1,), (1,)), ((), ())),
            preferred_element_type=jnp.float32)
        k0_scr[...] += part
        acc_ref[0] += jnp.sum(maskf)

        @pl.when((b == B - 1) & (t == NT - 1))
        def _finalize_k0():
            k0 = k0_scr[...]
            nrm = jnp.sqrt(jnp.sum(k0 * k0, axis=1, keepdims=True))
            k0n_scr[...] = (k0 / jnp.maximum(nrm, EPS)).astype(jnp.bfloat16)

    @pl.when(phase == 1)
    def _phase1():
        fnb = fn_scr[blk]
        k0n = k0n_scr[...]
        sim = jax.lax.dot_general(
            k0n, fnb,
            dimension_numbers=(((1,), (0,)), ((), ())),
            preferred_element_type=jnp.float32) * (1.0 / TAU)
        mx = jnp.max(sim, axis=0, keepdims=True)
        lse = mx + jnp.log(jnp.sum(jnp.exp(sim - mx), axis=0, keepdims=True))
        m = jnp.sum(maskf, axis=0, keepdims=True)
        part = jnp.sum(maskf * sim) - jnp.sum(m * lse)
        acc_ref[1] += part

        @pl.when((b == B - 1) & (t == NT - 1))
        def _final():
            out_ref[...] = jnp.broadcast_to(-acc_ref[1] / acc_ref[0], (1, 1))


def kernel(feat, gt):
    feat2 = feat.reshape(B, C, HW)
    gt2 = gt.reshape(B, K, HW)
    out = pl.pallas_call(
        _body,
        grid=(2, B, NT),
        in_specs=[
            pl.BlockSpec(
                (1, C, T_PX),
                lambda p, b, t: (jnp.where(p == 0, b, B - 1), 0,
                                 jnp.where(p == 0, t, NT - 1))),
            pl.BlockSpec((1, K, T_PX), lambda p, b, t: (b, 0, t)),
        ],
        out_specs=pl.BlockSpec((1, 1), lambda p, b, t: (0, 0)),
        out_shape=jax.ShapeDtypeStruct((1, 1), jnp.float32),
        scratch_shapes=[
            pltpu.VMEM((B * NT, C, T_PX), jnp.bfloat16),
            pltpu.VMEM((K, C), jnp.float32),
            pltpu.VMEM((K, C), jnp.bfloat16),
            pltpu.SMEM((2,), jnp.float32),
        ],
    )(feat2, gt2)
    return out.reshape(1)


# speedup vs baseline: 1.2228x; 1.2228x over previous
import jax
import jax.numpy as jnp
from jax.experimental import pallas as pl
from jax.experimental.pallas import tpu as pltpu

TAU = 0.07
EPS = 1e-12

B = 4
C = 512
K = 21
HW = 64 * 64
T_PX = 4096
NT = HW // T_PX


def _body(feat_ref, gt_ref, out_ref, fn_scr, k0_scr, k0n_scr, acc_ref):
    phase = pl.program_id(0)
    b = pl.program_id(1)
    t = pl.program_id(2)
    blk = b * NT + t

    @pl.when((phase == 0) & (b == 0) & (t == 0))
    def _init():
        k0_scr[...] = jnp.zeros_like(k0_scr)
        acc_ref[0] = 0.0
        acc_ref[1] = 0.0

    mask = (gt_ref[0] == 1)
    maskf = mask.astype(jnp.float32)

    @pl.when(phase == 0)
    def _phase0():
        x = feat_ref[0]
        s2 = jnp.sum(x * x, axis=0, keepdims=True)
        rnorm = 1.0 / jnp.maximum(jnp.sqrt(s2), EPS)
        fnb = (x * rnorm).astype(jnp.bfloat16)
        fn_scr[pl.ds(blk, 1)] = fnb[None]
        part = jax.lax.dot_general(
            maskf.astype(jnp.bfloat16), fnb,
            dimension_numbers=(((1,), (1,)), ((), ())),
            preferred_element_type=jnp.float32)
        k0_scr[...] += part
        acc_ref[0] += jnp.sum(maskf)

        @pl.when((b == B - 1) & (t == NT - 1))
        def _finalize_k0():
            k0 = k0_scr[...]
            nrm = jnp.sqrt(jnp.sum(k0 * k0, axis=1, keepdims=True))
            k0n_scr[...] = (k0 / jnp.maximum(nrm, EPS)).astype(jnp.bfloat16)

    @pl.when(phase == 1)
    def _phase1():
        fnb = fn_scr[blk]
        k0n = k0n_scr[...]
        sim = jax.lax.dot_general(
            k0n, fnb,
            dimension_numbers=(((1,), (0,)), ((), ())),
            preferred_element_type=jnp.float32) * (1.0 / TAU)
        mx = jnp.max(sim, axis=0, keepdims=True)
        lse = mx + jnp.log(jnp.sum(jnp.exp(sim - mx), axis=0, keepdims=True))
        m = jnp.sum(maskf, axis=0, keepdims=True)
        part = jnp.sum(maskf * sim) - jnp.sum(m * lse)
        acc_ref[1] += part

        @pl.when((b == B - 1) & (t == NT - 1))
        def _final():
            out_ref[...] = jnp.broadcast_to(-acc_ref[1] / acc_ref[0], (1, 1))


def kernel(feat, gt):
    feat2 = feat.reshape(B, C, HW)
    gt2 = gt.reshape(B, K, HW)
    out = pl.pallas_call(
        _body,
        grid=(2, B, NT),
        in_specs=[
            pl.BlockSpec(
                (1, C, T_PX),
                lambda p, b, t: (jnp.where(p == 0, b, B - 1), 0,
                                 jnp.where(p == 0, t, NT - 1))),
            pl.BlockSpec((1, K, T_PX), lambda p, b, t: (b, 0, t)),
        ],
        out_specs=pl.BlockSpec((1, 1), lambda p, b, t: (0, 0)),
        out_shape=jax.ShapeDtypeStruct((1, 1), jnp.float32),
        scratch_shapes=[
            pltpu.VMEM((B * NT, C, T_PX), jnp.bfloat16),
            pltpu.VMEM((K, C), jnp.float32),
            pltpu.VMEM((K, C), jnp.bfloat16),
            pltpu.SMEM((2,), jnp.float32),
        ],
    )(feat2, gt2)
    return out.reshape(1)
